# baseline (speedup 1.0000x reference)
import jax
import jax.numpy as jnp
import numpy as np
from jax import lax
from jax.experimental import pallas as pl
from jax.experimental.pallas import tpu as pltpu
from jax.experimental.pallas import tpu_sc as plsc

EMB = 32
CONT = 13
CATN = 26
CAT_DIMS = [100000] * CATN
CAT_OFFS = np.cumsum([0] + CAT_DIMS[:-1]).astype(np.int32)
BATCH = 16384
NW = 32
SB = BATCH // NW
GCH = 128


def _tok_kernel(xi_hbm, xc_hbm, w_hbm, b_hbm, tab_hbm, out_hbm,
                idx_v, rows_v, xc_v, wv, bv, gsem):
    wid = lax.axis_index("s") * 2 + lax.axis_index("c")
    b0 = wid * SB

    pltpu.sync_copy(w_hbm, wv)
    pltpu.sync_copy(b_hbm, bv)
    pltpu.sync_copy(xc_hbm.at[:, pl.ds(b0, SB)], xc_v)

    for j in range(CONT + 1):
        wj0 = wv[j, pl.ds(0, 16)]
        wj1 = wv[j, pl.ds(16, 16)]
        bj0 = bv[j, pl.ds(0, 16)]
        bj1 = bv[j, pl.ds(16, 16)]

        def dense_body(k, _, j=j, wj0=wj0, wj1=wj1, bj0=bj0, bj1=bj1):
            xcv = xc_v[j, pl.ds(k * 16, 16)]
            for t in range(16):
                s = xcv[t]
                r = k * 16 + t
                rows_v[r, pl.ds(0, 16)] = wj0 * s + bj0
                rows_v[r, pl.ds(16, 16)] = wj1 * s + bj1
            return 0

        lax.fori_loop(0, SB // 16, dense_body, 0)
        pltpu.sync_copy(rows_v, out_hbm.at[pl.ds(b0, SB), j])

    for c in range(CATN):
        for q in range(SB // GCH):
            pltpu.sync_copy(xi_hbm.at[c, pl.ds(b0 + q * GCH, GCH)], idx_v.at[q])
        off = int(CAT_OFFS[c])

        def off_body(k, _, off=off):
            q = k // (GCH // 16)
            r = (k % (GCH // 16)) * 16
            idx_v[q, pl.ds(r, 16)] = idx_v[q, pl.ds(r, 16)] + off
            return 0

        lax.fori_loop(0, SB // 16, off_body, 0)

        cps = []
        for q in range(SB // GCH):
            cps.append(pltpu.async_copy(
                tab_hbm.at[idx_v.at[q]],
                rows_v.at[pl.ds(q * GCH, GCH)], gsem))
        for cp in cps:
            cp.wait()

        bj0 = bv[14 + c, pl.ds(0, 16)]
        bj1 = bv[14 + c, pl.ds(16, 16)]

        def bias_body(i, _, bj0=bj0, bj1=bj1):
            rows_v[i, pl.ds(0, 16)] = rows_v[i, pl.ds(0, 16)] + bj0
            rows_v[i, pl.ds(16, 16)] = rows_v[i, pl.ds(16, 16)] + bj1
            return 0

        lax.fori_loop(0, SB, bias_body, 0)
        pltpu.sync_copy(rows_v, out_hbm.at[pl.ds(b0, SB), 14 + c])


def kernel(x, weight, bias, cat_table):
    xi_t = x[:, :CATN].astype(jnp.int32).T
    ones = jnp.ones((x.shape[0], 1), dtype=x.dtype)
    xc_t = jnp.concatenate([ones, x[:, CATN:]], axis=1).T
    bias_full = jnp.concatenate(
        [jnp.zeros((1, bias.shape[1]), dtype=bias.dtype), bias], axis=0)

    mesh = plsc.VectorSubcoreMesh(core_axis_name="c", subcore_axis_name="s")
    out = pl.kernel(
        _tok_kernel,
        out_type=jax.ShapeDtypeStruct((BATCH, CONT + 1 + CATN, EMB), jnp.float32),
        mesh=mesh,
        compiler_params=pltpu.CompilerParams(use_tc_tiling_on_sc=False),
        scratch_types=[
            pltpu.VMEM((SB // GCH, GCH), jnp.int32),
            pltpu.VMEM((SB, EMB), jnp.float32),
            pltpu.VMEM((CONT + 1, SB), jnp.float32),
            pltpu.VMEM((CONT + 1, EMB), jnp.float32),
            pltpu.VMEM((CONT + 1 + CATN, EMB), jnp.float32),
            pltpu.SemaphoreType.DMA,
        ],
    )(xi_t, xc_t, weight, bias_full, cat_table)
    return out
